# baseline (speedup 1.0000x reference)
import functools

import jax
import jax.numpy as jnp
from jax import lax
from jax.experimental import pallas as pl
from jax.experimental.pallas import tpu as pltpu
from jax.experimental.pallas import tpu_sc as plsc

BATCH = 16384
F_CAT = 26
F_CON = 13
D = 64
CAT_DIM = 100000
NC, NS, L = 2, 16, 16
NW = NC * NS
BW = BATCH // NW
NV = D // L

_mesh = plsc.VectorSubcoreMesh(core_axis_name="c", subcore_axis_name="s")


@functools.partial(
    pl.kernel,
    out_type=jax.ShapeDtypeStruct((BATCH, F_CAT + F_CON, D), jnp.float32),
    mesh=_mesh,
    scratch_types=[
        pltpu.VMEM((BW,), jnp.int32),
        pltpu.VMEM((BW, D), jnp.float32),
        pltpu.VMEM((BW,), jnp.float32),
        pltpu.VMEM((F_CAT, D), jnp.float32),
        pltpu.VMEM((F_CON, D), jnp.float32),
        pltpu.VMEM((F_CON, D), jnp.float32),
        pltpu.SemaphoreType.DMA,
    ],
    compiler_params=pltpu.CompilerParams(use_tc_tiling_on_sc=False),
)
def _fusion_tokenizer(cat_t_hbm, con_t_hbm, emb_hbm, cat_bias_hbm,
                      num_w_hbm, num_b_hbm, out_hbm,
                      idx_v, rows_v, con_v, bias_v, w_v, nb_v, sem):
    wid = lax.axis_index("s") * NC + lax.axis_index("c")
    base = wid * BW

    pltpu.sync_copy(cat_bias_hbm, bias_v)
    pltpu.sync_copy(num_w_hbm, w_v)
    pltpu.sync_copy(num_b_hbm, nb_v)

    @pl.loop(0, F_CAT)
    def _cat_field(f):
        pltpu.sync_copy(cat_t_hbm.at[f, pl.ds(base, BW)], idx_v)
        off = f * CAT_DIM

        @pl.loop(0, BW // L)
        def _add_off(p):
            idx_v[pl.ds(p * L, L)] = idx_v[pl.ds(p * L, L)] + off

        pltpu.async_copy(emb_hbm.at[idx_v], rows_v, sem).wait()

        bregs = [bias_v[f, pl.ds(q * L, L)] for q in range(NV)]

        @pl.loop(0, BW, unroll=4)
        def _bias_add(r):
            for q in range(NV):
                rows_v[r, pl.ds(q * L, L)] = rows_v[r, pl.ds(q * L, L)] + bregs[q]

        pltpu.sync_copy(rows_v, out_hbm.at[pl.ds(base, BW), f])

    @pl.loop(0, F_CON)
    def _con_field(j):
        pltpu.sync_copy(con_t_hbm.at[j, pl.ds(base, BW)], con_v)
        wregs = [w_v[j, pl.ds(q * L, L)] for q in range(NV)]
        bregs = [nb_v[j, pl.ds(q * L, L)] for q in range(NV)]

        @pl.loop(0, BW // L)
        def _rowgroup(g):
            v16 = con_v[pl.ds(g * L, L)]
            for l in range(L):
                s = v16[l]
                r = g * L + l
                for q in range(NV):
                    rows_v[r, pl.ds(q * L, L)] = wregs[q] * s + bregs[q]

        pltpu.sync_copy(rows_v, out_hbm.at[pl.ds(base, BW), F_CAT + j])


def kernel(anchor_cat, anchor_con, emb_table, cat_bias, num_weight, num_bias):
    cat_t = anchor_cat.T
    con_t = anchor_con.T
    out = _fusion_tokenizer(cat_t, con_t, emb_table, cat_bias,
                            num_weight, num_bias)
    return out.reshape(BATCH, (F_CAT + F_CON) * D)


# speedup vs baseline: 1.0412x; 1.0412x over previous
import functools

import jax
import jax.numpy as jnp
from jax import lax
from jax.experimental import pallas as pl
from jax.experimental.pallas import tpu as pltpu
from jax.experimental.pallas import tpu_sc as plsc

BATCH = 16384
F_CAT = 26
F_CON = 13
D = 64
CAT_DIM = 100000
NC, NS, L = 2, 16, 16
NW = NC * NS
BW = BATCH // NW
NV = D // L
NSLOT = 3

_mesh = plsc.VectorSubcoreMesh(core_axis_name="c", subcore_axis_name="s")


@functools.partial(
    pl.kernel,
    out_type=jax.ShapeDtypeStruct((BATCH, F_CAT + F_CON, D), jnp.float32),
    mesh=_mesh,
    scratch_types=[
        pltpu.VMEM((F_CAT, BW), jnp.int32),
        pltpu.VMEM((F_CON, BW), jnp.float32),
        [pltpu.VMEM((BW, D), jnp.float32) for _ in range(NSLOT)],
        pltpu.VMEM((F_CAT, D), jnp.float32),
        pltpu.VMEM((F_CON, D), jnp.float32),
        pltpu.VMEM((F_CON, D), jnp.float32),
        [pltpu.SemaphoreType.DMA for _ in range(NSLOT)],
        [pltpu.SemaphoreType.DMA for _ in range(NSLOT)],
    ],
    compiler_params=pltpu.CompilerParams(use_tc_tiling_on_sc=False),
)
def _fusion_tokenizer(cat_t_hbm, con_t_hbm, emb_hbm, cat_bias_hbm,
                      num_w_hbm, num_b_hbm, out_hbm,
                      idx_all, con_all, rows, bias_v, w_v, nb_v, gsem, ssem):
    wid = lax.axis_index("s") * NC + lax.axis_index("c")
    base = wid * BW

    pltpu.sync_copy(cat_bias_hbm, bias_v)
    pltpu.sync_copy(num_w_hbm, w_v)
    pltpu.sync_copy(num_b_hbm, nb_v)
    pltpu.sync_copy(cat_t_hbm.at[:, pl.ds(base, BW)], idx_all)
    pltpu.sync_copy(con_t_hbm.at[:, pl.ds(base, BW)], con_all)

    @pl.loop(0, F_CAT)
    def _field_off(f):
        off = f * CAT_DIM

        @pl.loop(0, BW // L, unroll=4)
        def _add_off(p):
            idx_all[f, pl.ds(p * L, L)] = idx_all[f, pl.ds(p * L, L)] + off

    def fire_gather(f):
        s = f % NSLOT
        pltpu.async_copy(emb_hbm.at[idx_all.at[f]], rows[s], gsem[s])

    def wait_gather(f):
        s = f % NSLOT
        pltpu.make_async_copy(emb_hbm.at[idx_all.at[f]], rows[s], gsem[s]).wait()

    def store_dst(f):
        return out_hbm.at[pl.ds(base, BW), f]

    def fire_store(f):
        s = f % NSLOT
        pltpu.async_copy(rows[s], store_dst(f), ssem[s])

    def wait_store(f):
        s = f % NSLOT
        pltpu.make_async_copy(rows[s], store_dst(f), ssem[s]).wait()

    for f in range(NSLOT):
        fire_gather(f)

    for f in range(F_CAT):
        s = f % NSLOT
        wait_gather(f)
        bregs = [bias_v[f, pl.ds(q * L, L)] for q in range(NV)]

        @pl.loop(0, BW, unroll=4)
        def _bias_add(r):
            for q in range(NV):
                rows[s][r, pl.ds(q * L, L)] = rows[s][r, pl.ds(q * L, L)] + bregs[q]

        fire_store(f)
        if f + NSLOT < F_CAT:
            wait_store(f)
            fire_gather(f + NSLOT)

    for j in range(F_CON):
        f = F_CAT + j
        s = f % NSLOT
        wait_store(f - NSLOT)
        wregs = [w_v[j, pl.ds(q * L, L)] for q in range(NV)]
        bregs = [nb_v[j, pl.ds(q * L, L)] for q in range(NV)]

        @pl.loop(0, BW // L)
        def _rowgroup(g):
            v16 = con_all[j, pl.ds(g * L, L)]
            for l in range(L):
                sc = v16[l]
                r = g * L + l
                for q in range(NV):
                    rows[s][r, pl.ds(q * L, L)] = wregs[q] * sc + bregs[q]

        fire_store(f)

    for f in range(F_CAT + F_CON - NSLOT, F_CAT + F_CON):
        wait_store(f)


def kernel(anchor_cat, anchor_con, emb_table, cat_bias, num_weight, num_bias):
    cat_t = anchor_cat.T
    con_t = anchor_con.T
    out = _fusion_tokenizer(cat_t, con_t, emb_table, cat_bias,
                            num_weight, num_bias)
    return out.reshape(BATCH, (F_CAT + F_CON) * D)
